# baseline (speedup 1.0000x reference)
import functools

import jax
import jax.numpy as jnp
from jax import lax
from jax.experimental import pallas as pl
from jax.experimental.pallas import tpu as pltpu
from jax.experimental.pallas import tpu_sc as plsc

N = 10000
E = 320000
F_IN = 128
HID = 128
NHID = 64
NCLS = 8
NGRAPH = 16

NC = 2
NS = 16
NW = NC * NS
EPT = E // NW
CH = 80
NCHUNK = EPT // CH
ROWS_PT = N // NS
DEN_PAD = NS * 640

BR = 1000
GRID = N // BR

_HIGH = lax.Precision.HIGHEST

_SELU_L = 1.0507009873554805
_SELU_A = 1.6732632423543772


def _selu(v):
    return _SELU_L * jnp.where(v > 0, v, _SELU_A * (jnp.exp(v) - 1.0))


def _head_block(h, att_s, att_d):
    a_s = jnp.dot(h, att_s, precision=_HIGH)
    a_d = jnp.dot(h, att_d, precision=_HIGH)
    z = a_s + a_d
    w_self = jnp.exp(jnp.maximum(z, 0.2 * z))
    return a_s, a_d, w_self


def _tc_head_body(x_ref, w_ref, att_s_ref, att_d_ref,
                  h_ref, num0_ref, asrc_ref, adst_ref, den0_ref):
    h = jnp.dot(x_ref[...], w_ref[...], precision=_HIGH)
    a_s, a_d, w_self = _head_block(h, att_s_ref[...], att_d_ref[...])
    h_ref[...] = h
    num0_ref[...] = h * w_self
    asrc_ref[...] = a_s
    adst_ref[...] = a_d
    den0_ref[...] = w_self


def _tc_head(x, W, att_s, att_d):
    f_in = x.shape[1]
    return pl.pallas_call(
        _tc_head_body,
        grid=(GRID,),
        in_specs=[
            pl.BlockSpec((BR, f_in), lambda i: (i, 0)),
            pl.BlockSpec((f_in, HID), lambda i: (0, 0)),
            pl.BlockSpec((HID, 1), lambda i: (0, 0)),
            pl.BlockSpec((HID, 1), lambda i: (0, 0)),
        ],
        out_specs=[
            pl.BlockSpec((BR, HID), lambda i: (i, 0)),
            pl.BlockSpec((BR, HID), lambda i: (i, 0)),
            pl.BlockSpec((BR, 1), lambda i: (i, 0)),
            pl.BlockSpec((BR, 1), lambda i: (i, 0)),
            pl.BlockSpec((BR, 1), lambda i: (i, 0)),
        ],
        out_shape=[
            jax.ShapeDtypeStruct((N, HID), jnp.float32),
            jax.ShapeDtypeStruct((N, HID), jnp.float32),
            jax.ShapeDtypeStruct((N, 1), jnp.float32),
            jax.ShapeDtypeStruct((N, 1), jnp.float32),
            jax.ShapeDtypeStruct((N, 1), jnp.float32),
        ],
    )(x, W, att_s, att_d)


def _tc_mid_body(nump_ref, denp_ref, num0_ref, den0_ref, b_ref,
                 w_ref, att_s_ref, att_d_ref,
                 h_ref, num0o_ref, asrc_ref, adst_ref, den0o_ref):
    nump = nump_ref[...]
    denp = denp_ref[...]
    num = num0_ref[...] + nump[0] + nump[1]
    den = den0_ref[...] + denp[0] + denp[1] + 1e-16
    h_prev = _selu(num / den + b_ref[...])
    h = jnp.dot(h_prev, w_ref[...], precision=_HIGH)
    a_s, a_d, w_self = _head_block(h, att_s_ref[...], att_d_ref[...])
    h_ref[...] = h
    num0o_ref[...] = h * w_self
    asrc_ref[...] = a_s
    adst_ref[...] = a_d
    den0o_ref[...] = w_self


def _tc_mid(numP, denP, num0, den0, b, W, att_s, att_d):
    return pl.pallas_call(
        _tc_mid_body,
        grid=(GRID,),
        in_specs=[
            pl.BlockSpec((NC, BR, HID), lambda i: (0, i, 0)),
            pl.BlockSpec((NC, BR, 1), lambda i: (0, i, 0)),
            pl.BlockSpec((BR, HID), lambda i: (i, 0)),
            pl.BlockSpec((BR, 1), lambda i: (i, 0)),
            pl.BlockSpec((1, HID), lambda i: (0, 0)),
            pl.BlockSpec((HID, HID), lambda i: (0, 0)),
            pl.BlockSpec((HID, 1), lambda i: (0, 0)),
            pl.BlockSpec((HID, 1), lambda i: (0, 0)),
        ],
        out_specs=[
            pl.BlockSpec((BR, HID), lambda i: (i, 0)),
            pl.BlockSpec((BR, HID), lambda i: (i, 0)),
            pl.BlockSpec((BR, 1), lambda i: (i, 0)),
            pl.BlockSpec((BR, 1), lambda i: (i, 0)),
            pl.BlockSpec((BR, 1), lambda i: (i, 0)),
        ],
        out_shape=[
            jax.ShapeDtypeStruct((N, HID), jnp.float32),
            jax.ShapeDtypeStruct((N, HID), jnp.float32),
            jax.ShapeDtypeStruct((N, 1), jnp.float32),
            jax.ShapeDtypeStruct((N, 1), jnp.float32),
            jax.ShapeDtypeStruct((N, 1), jnp.float32),
        ],
    )(numP, denP, num0, den0, b, W, att_s, att_d)


def _tc_final_body(nump_ref, denp_ref, num0_ref, den0_ref, b_ref, batch_ref,
                   fc1w_ref, fc1b_ref, fc2w_ref, fc2b_ref,
                   out_ref, acc_g, acc_c):
    i = pl.program_id(0)

    @pl.when(i == 0)
    def _():
        acc_g[...] = jnp.zeros_like(acc_g)
        acc_c[...] = jnp.zeros_like(acc_c)

    nump = nump_ref[...]
    denp = denp_ref[...]
    num = num0_ref[...] + nump[0] + nump[1]
    den = den0_ref[...] + denp[0] + denp[1] + 1e-16
    h = _selu(num / den + b_ref[...])

    gids = lax.broadcasted_iota(jnp.int32, (BR, NGRAPH), 1)
    onehot = (batch_ref[...] == gids).astype(jnp.float32)
    gsum = lax.dot_general(onehot, h, (((0,), (0,)), ((), ())),
                           precision=_HIGH)
    cnt = lax.dot_general(onehot, jnp.ones((BR, 1), jnp.float32),
                          (((0,), (0,)), ((), ())), precision=_HIGH)
    acc_g[...] += gsum
    acc_c[...] += cnt

    @pl.when(i == GRID - 1)
    def _():
        g = _selu(acc_g[...] / jnp.maximum(acc_c[...], 1.0))
        z1 = _selu(jnp.dot(g, fc1w_ref[...], precision=_HIGH) + fc1b_ref[...])
        z2 = jnp.dot(z1, fc2w_ref[...], precision=_HIGH) + fc2b_ref[...]
        m = jnp.max(z2, axis=-1, keepdims=True)
        zm = z2 - m
        out_ref[...] = zm - jnp.log(jnp.sum(jnp.exp(zm), axis=-1,
                                            keepdims=True))


def _tc_final(numP, denP, num0, den0, b, batch2d, fc1_W, fc1_b, fc2_W, fc2_b):
    return pl.pallas_call(
        _tc_final_body,
        grid=(GRID,),
        in_specs=[
            pl.BlockSpec((NC, BR, HID), lambda i: (0, i, 0)),
            pl.BlockSpec((NC, BR, 1), lambda i: (0, i, 0)),
            pl.BlockSpec((BR, HID), lambda i: (i, 0)),
            pl.BlockSpec((BR, 1), lambda i: (i, 0)),
            pl.BlockSpec((1, HID), lambda i: (0, 0)),
            pl.BlockSpec((BR, 1), lambda i: (i, 0)),
            pl.BlockSpec((HID, NHID), lambda i: (0, 0)),
            pl.BlockSpec((1, NHID), lambda i: (0, 0)),
            pl.BlockSpec((NHID, NCLS), lambda i: (0, 0)),
            pl.BlockSpec((1, NCLS), lambda i: (0, 0)),
        ],
        out_specs=pl.BlockSpec((NGRAPH, NCLS), lambda i: (0, 0)),
        out_shape=jax.ShapeDtypeStruct((NGRAPH, NCLS), jnp.float32),
        scratch_shapes=[
            pltpu.VMEM((NGRAPH, HID), jnp.float32),
            pltpu.VMEM((NGRAPH, 1), jnp.float32),
        ],
    )(numP, denP, num0, den0, b, batch2d, fc1_W, fc1_b, fc2_W, fc2_b)


NBUF = 2
H1 = 64
H2 = NCHUNK - H1
HMAX = max(H1, H2)


def _sc_edge_body(h_hbm, asrc_hbm, adst_hbm, src_hbm, dst_hbm,
                  nump_hbm, denp_hbm,
                  num_sh, den_sh,
                  idx_s, idx_d, sv, dv, wv, rows, zden,
                  *sems):
    c = lax.axis_index("c")
    s = lax.axis_index("s")

    zrow = rows.at[0]

    def _zrow_body(r, _):
        for j in range(HID // 16):
            zrow[r, pl.ds(j * 16, 16)] = jnp.zeros((16,), jnp.float32)
        return 0

    lax.fori_loop(0, CH, _zrow_body, 0)
    for j in range(640 // 16):
        zden[pl.ds(j * 16, 16)] = jnp.zeros((16,), jnp.float32)

    row0 = s * ROWS_PT
    for k in range(ROWS_PT // CH):
        pltpu.sync_copy(zrow, num_sh.at[pl.ds(row0 + k * CH, CH)])
    rem = ROWS_PT % CH
    if rem:
        pltpu.sync_copy(zrow.at[pl.ds(0, rem)],
                        num_sh.at[pl.ds(row0 + (ROWS_PT // CH) * CH, rem)])
    pltpu.sync_copy(zden, den_sh.at[pl.ds(s * 640, 640)])

    w = c * NS + s
    for p in range(H2, HMAX + NBUF):
        for j in range(CH // 16):
            idx_s[p, pl.ds(j * 16, 16)] = jnp.zeros((16,), jnp.int32)
            idx_d[p, pl.ds(j * 16, 16)] = jnp.zeros((16,), jnp.int32)
    plsc.subcore_barrier()

    def _start(b, k):
        is_k = idx_s.at[k]
        pltpu.async_copy(asrc_hbm.at[is_k], sv.at[b], sems[3 * b])
        pltpu.async_copy(adst_hbm.at[idx_d.at[k]], dv.at[b], sems[3 * b + 1])
        pltpu.async_copy(h_hbm.at[is_k], rows.at[b], sems[3 * b + 2])

    def _drain(b, k):
        is_k = idx_s.at[k]
        pltpu.make_async_copy(asrc_hbm.at[is_k], sv.at[b],
                              sems[3 * b]).wait()
        pltpu.make_async_copy(adst_hbm.at[idx_d.at[k]], dv.at[b],
                              sems[3 * b + 1]).wait()
        pltpu.make_async_copy(h_hbm.at[is_k], rows.at[b],
                              sems[3 * b + 2]).wait()

    def _process(b, k):
        id_k = idx_d.at[k]
        _drain(b, k)
        ws = []
        for g in range(CH // 16):
            z = sv[b, pl.ds(g * 16, 16)] + dv[b, pl.ds(g * 16, 16)]
            wq = jnp.exp(jnp.maximum(z, 0.2 * z))
            wv[pl.ds(g * 16, 16)] = wq
            ws.append(wq)
        for g in range(CH // 16):
            for i in range(16):
                wi = ws[g][i]
                r = g * 16 + i
                for j in range(HID // 16):
                    rows[b, r, pl.ds(j * 16, 16)] = (
                        rows[b, r, pl.ds(j * 16, 16)] * wi)
        pltpu.sync_copy(rows.at[b], num_sh.at[id_k], add=True)
        pltpu.sync_copy(wv, den_sh.at[id_k], add=True)

    def _ring(g, _):
        for b in range(NBUF):
            k = g * NBUF + b
            _process(b, k)
            _start(b, k + NBUF)
        return 0

    for h0, hn in ((0, H1), (H1, H2)):
        pltpu.sync_copy(src_hbm.at[w].at[pl.ds(h0, hn)],
                        idx_s.at[pl.ds(0, hn)])
        pltpu.sync_copy(dst_hbm.at[w].at[pl.ds(h0, hn)],
                        idx_d.at[pl.ds(0, hn)])
        pairs = hn // NBUF
        tail = hn - pairs * NBUF
        for b in range(NBUF):
            _start(b, b)
        lax.fori_loop(0, pairs, _ring, 0)
        for t in range(tail):
            _process(t, pairs * NBUF + t)
        for t in range(tail, NBUF):
            _drain(t, pairs * NBUF + t)
    plsc.subcore_barrier()

    woff = s * 624
    pltpu.sync_copy(num_sh.at[pl.ds(woff, 624)],
                    nump_hbm.at[c].at[pl.ds(woff, 624)])
    pltpu.sync_copy(den_sh.at[pl.ds(s * 640, 640)],
                    denp_hbm.at[c].at[pl.ds(s * 640, 640)])

    @pl.when(s == NS - 1)
    def _():
        pltpu.sync_copy(num_sh.at[pl.ds(NS * 624, N - NS * 624)],
                        nump_hbm.at[c].at[pl.ds(NS * 624, N - NS * 624)])


def _sc_edge(h, asrc, adst, src, dst):
    mesh = plsc.VectorSubcoreMesh(core_axis_name="c", subcore_axis_name="s",
                                  num_cores=NC, num_subcores=NS)
    f = pl.kernel(
        _sc_edge_body,
        out_type=[
            jax.ShapeDtypeStruct((NC, N, HID), jnp.float32),
            jax.ShapeDtypeStruct((NC, DEN_PAD), jnp.float32),
        ],
        mesh=mesh,
        scratch_types=[
            pltpu.VMEM_SHARED((N, HID), jnp.float32),
            pltpu.VMEM_SHARED((DEN_PAD,), jnp.float32),
            pltpu.VMEM((HMAX + NBUF, CH), jnp.int32),
            pltpu.VMEM((HMAX + NBUF, CH), jnp.int32),
            pltpu.VMEM((NBUF, CH), jnp.float32),
            pltpu.VMEM((NBUF, CH), jnp.float32),
            pltpu.VMEM((CH,), jnp.float32),
            pltpu.VMEM((NBUF, CH, HID), jnp.float32),
            pltpu.VMEM((640,), jnp.float32),
        ] + [pltpu.SemaphoreType.DMA] * (3 * NBUF),
    )
    return f(h, asrc, adst, src, dst)


def kernel(x, edge_index, batch, W1, att_src1, att_dst1, b1,
           W2, att_src2, att_dst2, b2, fc1_W, fc1_b, fc2_W, fc2_b):
    src = edge_index[0].reshape(NW, NCHUNK, CH)
    dst = edge_index[1].reshape(NW, NCHUNK, CH)

    h1, num01, asrc1, adst1, den01 = _tc_head(
        x, W1, att_src1.reshape(HID, 1), att_dst1.reshape(HID, 1))
    numP1, denP1 = _sc_edge(h1, asrc1.reshape(N), adst1.reshape(N), src, dst)
    denP1 = denP1[:, :N].reshape(NC, N, 1)

    h2, num02, asrc2, adst2, den02 = _tc_mid(
        numP1, denP1, num01, den01, b1.reshape(1, HID),
        W2, att_src2.reshape(HID, 1), att_dst2.reshape(HID, 1))
    numP2, denP2 = _sc_edge(h2, asrc2.reshape(N), adst2.reshape(N), src, dst)
    denP2 = denP2[:, :N].reshape(NC, N, 1)

    return _tc_final(
        numP2, denP2, num02, den02, b2.reshape(1, HID),
        batch.reshape(N, 1), fc1_W, fc1_b.reshape(1, NHID),
        fc2_W, fc2_b.reshape(1, NCLS))


# speedup vs baseline: 1.2532x; 1.2532x over previous
import functools

import jax
import jax.numpy as jnp
from jax import lax
from jax.experimental import pallas as pl
from jax.experimental.pallas import tpu as pltpu
from jax.experimental.pallas import tpu_sc as plsc

N = 10000
E = 320000
F_IN = 128
HID = 128
NHID = 64
NCLS = 8
NGRAPH = 16

NC = 2
NS = 16
NW = NC * NS
EPT = E // NW
CH = 80
NCHUNK = EPT // CH
ROWS_PT = N // NS
DEN_PAD = NS * 640

BR = 1000
GRID = N // BR

_HIGH = lax.Precision.HIGHEST

_SELU_L = 1.0507009873554805
_SELU_A = 1.6732632423543772


def _selu(v):
    return _SELU_L * jnp.where(v > 0, v, _SELU_A * (jnp.exp(v) - 1.0))


def _head_block(h, att_s, att_d):
    a_s = jnp.dot(h, att_s, precision=_HIGH)
    a_d = jnp.dot(h, att_d, precision=_HIGH)
    z = a_s + a_d
    w_self = jnp.exp(jnp.maximum(z, 0.2 * z))
    return a_s, a_d, w_self


def _tc_head_body(x_ref, w_ref, att_s_ref, att_d_ref,
                  h_ref, num0_ref, asrc_ref, adst_ref, den0_ref):
    h = jnp.dot(x_ref[...], w_ref[...], precision=_HIGH)
    a_s, a_d, w_self = _head_block(h, att_s_ref[...], att_d_ref[...])
    h_ref[...] = h
    num0_ref[...] = h * w_self
    asrc_ref[...] = a_s
    adst_ref[...] = a_d
    den0_ref[...] = w_self


def _tc_head(x, W, att_s, att_d):
    f_in = x.shape[1]
    return pl.pallas_call(
        _tc_head_body,
        grid=(GRID,),
        in_specs=[
            pl.BlockSpec((BR, f_in), lambda i: (i, 0)),
            pl.BlockSpec((f_in, HID), lambda i: (0, 0)),
            pl.BlockSpec((HID, 1), lambda i: (0, 0)),
            pl.BlockSpec((HID, 1), lambda i: (0, 0)),
        ],
        out_specs=[
            pl.BlockSpec((BR, HID), lambda i: (i, 0)),
            pl.BlockSpec((BR, HID), lambda i: (i, 0)),
            pl.BlockSpec((BR, 1), lambda i: (i, 0)),
            pl.BlockSpec((BR, 1), lambda i: (i, 0)),
            pl.BlockSpec((BR, 1), lambda i: (i, 0)),
        ],
        out_shape=[
            jax.ShapeDtypeStruct((N, HID), jnp.float32),
            jax.ShapeDtypeStruct((N, HID), jnp.float32),
            jax.ShapeDtypeStruct((N, 1), jnp.float32),
            jax.ShapeDtypeStruct((N, 1), jnp.float32),
            jax.ShapeDtypeStruct((N, 1), jnp.float32),
        ],
    )(x, W, att_s, att_d)


def _tc_mid_body(nump_ref, denp_ref, num0_ref, den0_ref, b_ref,
                 w_ref, att_s_ref, att_d_ref,
                 h_ref, num0o_ref, asrc_ref, adst_ref, den0o_ref):
    nump = nump_ref[...]
    denp = denp_ref[...]
    num = num0_ref[...] + nump[0] + nump[1]
    den = den0_ref[...] + denp[0] + denp[1] + 1e-16
    h_prev = _selu(num / den + b_ref[...])
    h = jnp.dot(h_prev, w_ref[...], precision=_HIGH)
    a_s, a_d, w_self = _head_block(h, att_s_ref[...], att_d_ref[...])
    h_ref[...] = h
    num0o_ref[...] = h * w_self
    asrc_ref[...] = a_s
    adst_ref[...] = a_d
    den0o_ref[...] = w_self


def _tc_mid(numP, denP, num0, den0, b, W, att_s, att_d):
    return pl.pallas_call(
        _tc_mid_body,
        grid=(GRID,),
        in_specs=[
            pl.BlockSpec((NC, BR, HID), lambda i: (0, i, 0)),
            pl.BlockSpec((NC, BR, 1), lambda i: (0, i, 0)),
            pl.BlockSpec((BR, HID), lambda i: (i, 0)),
            pl.BlockSpec((BR, 1), lambda i: (i, 0)),
            pl.BlockSpec((1, HID), lambda i: (0, 0)),
            pl.BlockSpec((HID, HID), lambda i: (0, 0)),
            pl.BlockSpec((HID, 1), lambda i: (0, 0)),
            pl.BlockSpec((HID, 1), lambda i: (0, 0)),
        ],
        out_specs=[
            pl.BlockSpec((BR, HID), lambda i: (i, 0)),
            pl.BlockSpec((BR, HID), lambda i: (i, 0)),
            pl.BlockSpec((BR, 1), lambda i: (i, 0)),
            pl.BlockSpec((BR, 1), lambda i: (i, 0)),
            pl.BlockSpec((BR, 1), lambda i: (i, 0)),
        ],
        out_shape=[
            jax.ShapeDtypeStruct((N, HID), jnp.float32),
            jax.ShapeDtypeStruct((N, HID), jnp.float32),
            jax.ShapeDtypeStruct((N, 1), jnp.float32),
            jax.ShapeDtypeStruct((N, 1), jnp.float32),
            jax.ShapeDtypeStruct((N, 1), jnp.float32),
        ],
    )(numP, denP, num0, den0, b, W, att_s, att_d)


def _tc_final_body(nump_ref, denp_ref, num0_ref, den0_ref, b_ref, batch_ref,
                   fc1w_ref, fc1b_ref, fc2w_ref, fc2b_ref,
                   out_ref, acc_g, acc_c):
    i = pl.program_id(0)

    @pl.when(i == 0)
    def _():
        acc_g[...] = jnp.zeros_like(acc_g)
        acc_c[...] = jnp.zeros_like(acc_c)

    nump = nump_ref[...]
    denp = denp_ref[...]
    num = num0_ref[...] + nump[0] + nump[1]
    den = den0_ref[...] + denp[0] + denp[1] + 1e-16
    h = _selu(num / den + b_ref[...])

    gids = lax.broadcasted_iota(jnp.int32, (BR, NGRAPH), 1)
    onehot = (batch_ref[...] == gids).astype(jnp.float32)
    gsum = lax.dot_general(onehot, h, (((0,), (0,)), ((), ())),
                           precision=_HIGH)
    cnt = lax.dot_general(onehot, jnp.ones((BR, 1), jnp.float32),
                          (((0,), (0,)), ((), ())), precision=_HIGH)
    acc_g[...] += gsum
    acc_c[...] += cnt

    @pl.when(i == GRID - 1)
    def _():
        g = _selu(acc_g[...] / jnp.maximum(acc_c[...], 1.0))
        z1 = _selu(jnp.dot(g, fc1w_ref[...], precision=_HIGH) + fc1b_ref[...])
        z2 = jnp.dot(z1, fc2w_ref[...], precision=_HIGH) + fc2b_ref[...]
        m = jnp.max(z2, axis=-1, keepdims=True)
        zm = z2 - m
        out_ref[...] = zm - jnp.log(jnp.sum(jnp.exp(zm), axis=-1,
                                            keepdims=True))


def _tc_final(numP, denP, num0, den0, b, batch2d, fc1_W, fc1_b, fc2_W, fc2_b):
    return pl.pallas_call(
        _tc_final_body,
        grid=(GRID,),
        in_specs=[
            pl.BlockSpec((NC, BR, HID), lambda i: (0, i, 0)),
            pl.BlockSpec((NC, BR, 1), lambda i: (0, i, 0)),
            pl.BlockSpec((BR, HID), lambda i: (i, 0)),
            pl.BlockSpec((BR, 1), lambda i: (i, 0)),
            pl.BlockSpec((1, HID), lambda i: (0, 0)),
            pl.BlockSpec((BR, 1), lambda i: (i, 0)),
            pl.BlockSpec((HID, NHID), lambda i: (0, 0)),
            pl.BlockSpec((1, NHID), lambda i: (0, 0)),
            pl.BlockSpec((NHID, NCLS), lambda i: (0, 0)),
            pl.BlockSpec((1, NCLS), lambda i: (0, 0)),
        ],
        out_specs=pl.BlockSpec((NGRAPH, NCLS), lambda i: (0, 0)),
        out_shape=jax.ShapeDtypeStruct((NGRAPH, NCLS), jnp.float32),
        scratch_shapes=[
            pltpu.VMEM((NGRAPH, HID), jnp.float32),
            pltpu.VMEM((NGRAPH, 1), jnp.float32),
        ],
    )(numP, denP, num0, den0, b, batch2d, fc1_W, fc1_b, fc2_W, fc2_b)


def _sc_edge_body(h_hbm, asrc_hbm, adst_hbm, src_hbm, dst_hbm,
                  nump_hbm, denp_hbm,
                  num_sh, den_sh,
                  idx_s, idx_d, sv, dv, wv, rows, zden,
                  sem_s, sem_d, sem_r):
    c = lax.axis_index("c")
    s = lax.axis_index("s")

    def _zrow_body(r, _):
        for j in range(HID // 16):
            rows[r, pl.ds(j * 16, 16)] = jnp.zeros((16,), jnp.float32)
        return 0

    lax.fori_loop(0, CH, _zrow_body, 0)
    for j in range(640 // 16):
        zden[pl.ds(j * 16, 16)] = jnp.zeros((16,), jnp.float32)

    row0 = s * ROWS_PT
    for k in range(ROWS_PT // CH):
        pltpu.sync_copy(rows, num_sh.at[pl.ds(row0 + k * CH, CH)])
    rem = ROWS_PT % CH
    if rem:
        pltpu.sync_copy(rows.at[pl.ds(0, rem)],
                        num_sh.at[pl.ds(row0 + (ROWS_PT // CH) * CH, rem)])
    pltpu.sync_copy(zden, den_sh.at[pl.ds(s * 640, 640)])

    w = c * NS + s
    pltpu.sync_copy(src_hbm.at[w], idx_s)
    pltpu.sync_copy(dst_hbm.at[w], idx_d)
    plsc.subcore_barrier()

    def _chunk(k, _):
        is_k = idx_s.at[k]
        id_k = idx_d.at[k]
        cp_r = pltpu.async_copy(h_hbm.at[is_k], rows, sem_r)
        cp_s = pltpu.async_copy(asrc_hbm.at[is_k], sv, sem_s)
        cp_d = pltpu.async_copy(adst_hbm.at[id_k], dv, sem_d)
        cp_s.wait()
        cp_d.wait()
        wis = []
        for g in range(CH // 16):
            z = sv[pl.ds(g * 16, 16)] + dv[pl.ds(g * 16, 16)]
            wq = jnp.exp(jnp.maximum(z, 0.2 * z))
            wv[pl.ds(g * 16, 16)] = wq
            for i in range(16):
                wis.append(wq[i])
        cp_r.wait()
        for r in range(CH):
            wi = wis[r]
            for j in range(HID // 16):
                rows[r, pl.ds(j * 16, 16)] = rows[r, pl.ds(j * 16, 16)] * wi
        pltpu.sync_copy(rows, num_sh.at[id_k], add=True)
        pltpu.sync_copy(wv, den_sh.at[id_k], add=True)
        return 0

    lax.fori_loop(0, NCHUNK, _chunk, 0)
    plsc.subcore_barrier()

    woff = s * 624
    pltpu.sync_copy(num_sh.at[pl.ds(woff, 624)],
                    nump_hbm.at[c].at[pl.ds(woff, 624)])
    pltpu.sync_copy(den_sh.at[pl.ds(s * 640, 640)],
                    denp_hbm.at[c].at[pl.ds(s * 640, 640)])

    @pl.when(s == NS - 1)
    def _():
        pltpu.sync_copy(num_sh.at[pl.ds(NS * 624, N - NS * 624)],
                        nump_hbm.at[c].at[pl.ds(NS * 624, N - NS * 624)])


def _sc_edge(h, asrc, adst, src, dst):
    mesh = plsc.VectorSubcoreMesh(core_axis_name="c", subcore_axis_name="s",
                                  num_cores=NC, num_subcores=NS)
    f = pl.kernel(
        _sc_edge_body,
        out_type=[
            jax.ShapeDtypeStruct((NC, N, HID), jnp.float32),
            jax.ShapeDtypeStruct((NC, DEN_PAD), jnp.float32),
        ],
        mesh=mesh,
        scratch_types=[
            pltpu.VMEM_SHARED((N, HID), jnp.float32),
            pltpu.VMEM_SHARED((DEN_PAD,), jnp.float32),
            pltpu.VMEM((NCHUNK, CH), jnp.int32),
            pltpu.VMEM((NCHUNK, CH), jnp.int32),
            pltpu.VMEM((CH,), jnp.float32),
            pltpu.VMEM((CH,), jnp.float32),
            pltpu.VMEM((CH,), jnp.float32),
            pltpu.VMEM((CH, HID), jnp.float32),
            pltpu.VMEM((640,), jnp.float32),
            pltpu.SemaphoreType.DMA,
            pltpu.SemaphoreType.DMA,
            pltpu.SemaphoreType.DMA,
        ],
    )
    return f(h, asrc, adst, src, dst)


def kernel(x, edge_index, batch, W1, att_src1, att_dst1, b1,
           W2, att_src2, att_dst2, b2, fc1_W, fc1_b, fc2_W, fc2_b):
    src = edge_index[0].reshape(NW, NCHUNK, CH)
    dst = edge_index[1].reshape(NW, NCHUNK, CH)

    h1, num01, asrc1, adst1, den01 = _tc_head(
        x, W1, att_src1.reshape(HID, 1), att_dst1.reshape(HID, 1))
    numP1, denP1 = _sc_edge(h1, asrc1.reshape(N), adst1.reshape(N), src, dst)
    denP1 = denP1[:, :N].reshape(NC, N, 1)

    h2, num02, asrc2, adst2, den02 = _tc_mid(
        numP1, denP1, num01, den01, b1.reshape(1, HID),
        W2, att_src2.reshape(HID, 1), att_dst2.reshape(HID, 1))
    numP2, denP2 = _sc_edge(h2, asrc2.reshape(N), adst2.reshape(N), src, dst)
    denP2 = denP2[:, :N].reshape(NC, N, 1)

    return _tc_final(
        numP2, denP2, num02, den02, b2.reshape(1, HID),
        batch.reshape(N, 1), fc1_W, fc1_b.reshape(1, NHID),
        fc2_W, fc2_b.reshape(1, NCLS))


# speedup vs baseline: 1.3449x; 1.0731x over previous
import functools

import jax
import jax.numpy as jnp
from jax import lax
from jax.experimental import pallas as pl
from jax.experimental.pallas import tpu as pltpu
from jax.experimental.pallas import tpu_sc as plsc

N = 10000
E = 320000
F_IN = 128
HID = 128
NHID = 64
NCLS = 8
NGRAPH = 16

NC = 2
NS = 16
NW = NC * NS
EPT = E // NW
CH = 80
NCHUNK = EPT // CH
ROWS_PT = N // NS
DEN_PAD = NS * 640

BR = 1000
GRID = N // BR

_HIGH = lax.Precision.HIGHEST

_SELU_L = 1.0507009873554805
_SELU_A = 1.6732632423543772


def _selu(v):
    return _SELU_L * jnp.where(v > 0, v, _SELU_A * (jnp.exp(v) - 1.0))


def _head_block(h, att_s, att_d):
    a_s = jnp.dot(h, att_s, precision=_HIGH)
    a_d = jnp.dot(h, att_d, precision=_HIGH)
    z = a_s + a_d
    w_self = jnp.exp(jnp.maximum(z, 0.2 * z))
    return a_s, a_d, w_self


def _tc_head_body(x_ref, w_ref, att_s_ref, att_d_ref,
                  h_ref, num0_ref, asrc_ref, adst_ref, den0_ref):
    h = jnp.dot(x_ref[...], w_ref[...], precision=_HIGH)
    a_s, a_d, w_self = _head_block(h, att_s_ref[...], att_d_ref[...])
    h_ref[...] = h
    num0_ref[...] = h * w_self
    asrc_ref[...] = a_s
    adst_ref[...] = a_d
    den0_ref[...] = w_self


def _tc_head(x, W, att_s, att_d):
    f_in = x.shape[1]
    return pl.pallas_call(
        _tc_head_body,
        grid=(GRID,),
        in_specs=[
            pl.BlockSpec((BR, f_in), lambda i: (i, 0)),
            pl.BlockSpec((f_in, HID), lambda i: (0, 0)),
            pl.BlockSpec((HID, 1), lambda i: (0, 0)),
            pl.BlockSpec((HID, 1), lambda i: (0, 0)),
        ],
        out_specs=[
            pl.BlockSpec((BR, HID), lambda i: (i, 0)),
            pl.BlockSpec((BR, HID), lambda i: (i, 0)),
            pl.BlockSpec((BR, 1), lambda i: (i, 0)),
            pl.BlockSpec((BR, 1), lambda i: (i, 0)),
            pl.BlockSpec((BR, 1), lambda i: (i, 0)),
        ],
        out_shape=[
            jax.ShapeDtypeStruct((N, HID), jnp.float32),
            jax.ShapeDtypeStruct((N, HID), jnp.float32),
            jax.ShapeDtypeStruct((N, 1), jnp.float32),
            jax.ShapeDtypeStruct((N, 1), jnp.float32),
            jax.ShapeDtypeStruct((N, 1), jnp.float32),
        ],
    )(x, W, att_s, att_d)


def _tc_mid_body(nump_ref, denp_ref, num0_ref, den0_ref, b_ref,
                 w_ref, att_s_ref, att_d_ref,
                 h_ref, num0o_ref, asrc_ref, adst_ref, den0o_ref):
    nump = nump_ref[...]
    denp = denp_ref[...]
    num = num0_ref[...] + nump[0] + nump[1]
    den = den0_ref[...] + denp[0] + denp[1] + 1e-16
    h_prev = _selu(num / den + b_ref[...])
    h = jnp.dot(h_prev, w_ref[...], precision=_HIGH)
    a_s, a_d, w_self = _head_block(h, att_s_ref[...], att_d_ref[...])
    h_ref[...] = h
    num0o_ref[...] = h * w_self
    asrc_ref[...] = a_s
    adst_ref[...] = a_d
    den0o_ref[...] = w_self


def _tc_mid(numP, denP, num0, den0, b, W, att_s, att_d):
    return pl.pallas_call(
        _tc_mid_body,
        grid=(GRID,),
        in_specs=[
            pl.BlockSpec((NC, BR, HID), lambda i: (0, i, 0)),
            pl.BlockSpec((NC, BR, 1), lambda i: (0, i, 0)),
            pl.BlockSpec((BR, HID), lambda i: (i, 0)),
            pl.BlockSpec((BR, 1), lambda i: (i, 0)),
            pl.BlockSpec((1, HID), lambda i: (0, 0)),
            pl.BlockSpec((HID, HID), lambda i: (0, 0)),
            pl.BlockSpec((HID, 1), lambda i: (0, 0)),
            pl.BlockSpec((HID, 1), lambda i: (0, 0)),
        ],
        out_specs=[
            pl.BlockSpec((BR, HID), lambda i: (i, 0)),
            pl.BlockSpec((BR, HID), lambda i: (i, 0)),
            pl.BlockSpec((BR, 1), lambda i: (i, 0)),
            pl.BlockSpec((BR, 1), lambda i: (i, 0)),
            pl.BlockSpec((BR, 1), lambda i: (i, 0)),
        ],
        out_shape=[
            jax.ShapeDtypeStruct((N, HID), jnp.float32),
            jax.ShapeDtypeStruct((N, HID), jnp.float32),
            jax.ShapeDtypeStruct((N, 1), jnp.float32),
            jax.ShapeDtypeStruct((N, 1), jnp.float32),
            jax.ShapeDtypeStruct((N, 1), jnp.float32),
        ],
    )(numP, denP, num0, den0, b, W, att_s, att_d)


def _tc_final_body(nump_ref, denp_ref, num0_ref, den0_ref, b_ref, batch_ref,
                   fc1w_ref, fc1b_ref, fc2w_ref, fc2b_ref,
                   out_ref, acc_g, acc_c):
    i = pl.program_id(0)

    @pl.when(i == 0)
    def _():
        acc_g[...] = jnp.zeros_like(acc_g)
        acc_c[...] = jnp.zeros_like(acc_c)

    nump = nump_ref[...]
    denp = denp_ref[...]
    num = num0_ref[...] + nump[0] + nump[1]
    den = den0_ref[...] + denp[0] + denp[1] + 1e-16
    h = _selu(num / den + b_ref[...])

    gids = lax.broadcasted_iota(jnp.int32, (BR, NGRAPH), 1)
    onehot = (batch_ref[...] == gids).astype(jnp.float32)
    gsum = lax.dot_general(onehot, h, (((0,), (0,)), ((), ())),
                           precision=_HIGH)
    cnt = lax.dot_general(onehot, jnp.ones((BR, 1), jnp.float32),
                          (((0,), (0,)), ((), ())), precision=_HIGH)
    acc_g[...] += gsum
    acc_c[...] += cnt

    @pl.when(i == GRID - 1)
    def _():
        g = _selu(acc_g[...] / jnp.maximum(acc_c[...], 1.0))
        z1 = _selu(jnp.dot(g, fc1w_ref[...], precision=_HIGH) + fc1b_ref[...])
        z2 = jnp.dot(z1, fc2w_ref[...], precision=_HIGH) + fc2b_ref[...]
        m = jnp.max(z2, axis=-1, keepdims=True)
        zm = z2 - m
        out_ref[...] = zm - jnp.log(jnp.sum(jnp.exp(zm), axis=-1,
                                            keepdims=True))


def _tc_final(numP, denP, num0, den0, b, batch2d, fc1_W, fc1_b, fc2_W, fc2_b):
    return pl.pallas_call(
        _tc_final_body,
        grid=(GRID,),
        in_specs=[
            pl.BlockSpec((NC, BR, HID), lambda i: (0, i, 0)),
            pl.BlockSpec((NC, BR, 1), lambda i: (0, i, 0)),
            pl.BlockSpec((BR, HID), lambda i: (i, 0)),
            pl.BlockSpec((BR, 1), lambda i: (i, 0)),
            pl.BlockSpec((1, HID), lambda i: (0, 0)),
            pl.BlockSpec((BR, 1), lambda i: (i, 0)),
            pl.BlockSpec((HID, NHID), lambda i: (0, 0)),
            pl.BlockSpec((1, NHID), lambda i: (0, 0)),
            pl.BlockSpec((NHID, NCLS), lambda i: (0, 0)),
            pl.BlockSpec((1, NCLS), lambda i: (0, 0)),
        ],
        out_specs=pl.BlockSpec((NGRAPH, NCLS), lambda i: (0, 0)),
        out_shape=jax.ShapeDtypeStruct((NGRAPH, NCLS), jnp.float32),
        scratch_shapes=[
            pltpu.VMEM((NGRAPH, HID), jnp.float32),
            pltpu.VMEM((NGRAPH, 1), jnp.float32),
        ],
    )(numP, denP, num0, den0, b, batch2d, fc1_W, fc1_b, fc2_W, fc2_b)


H1 = 64
H2 = NCHUNK - H1


def _sc_edge_body(h_hbm, asrc_hbm, adst_hbm, src_hbm, dst_hbm,
                  nump_hbm, denp_hbm,
                  num_sh, den_sh,
                  idx_s, idx_d, sv, dv, wv, rows, rows_o, zden,
                  sem_s, sem_d, sem_r, sem_w0, sem_w1, sem_v0, sem_v1):
    c = lax.axis_index("c")
    s = lax.axis_index("s")
    sem_w = (sem_w0, sem_w1)
    sem_v = (sem_v0, sem_v1)

    def _zrow_body(r, _):
        for j in range(HID // 16):
            rows[r, pl.ds(j * 16, 16)] = jnp.zeros((16,), jnp.float32)
        return 0

    lax.fori_loop(0, CH, _zrow_body, 0)
    for j in range(640 // 16):
        zden[pl.ds(j * 16, 16)] = jnp.zeros((16,), jnp.float32)

    row0 = s * ROWS_PT
    for k in range(ROWS_PT // CH):
        pltpu.sync_copy(rows, num_sh.at[pl.ds(row0 + k * CH, CH)])
    rem = ROWS_PT % CH
    if rem:
        pltpu.sync_copy(rows.at[pl.ds(0, rem)],
                        num_sh.at[pl.ds(row0 + (ROWS_PT // CH) * CH, rem)])
    pltpu.sync_copy(zden, den_sh.at[pl.ds(s * 640, 640)])

    plsc.subcore_barrier()
    w = c * NS + s

    def _gather(k):
        is_k = idx_s.at[k]
        cp_r = pltpu.async_copy(h_hbm.at[is_k], rows, sem_r)
        cp_s = pltpu.async_copy(asrc_hbm.at[is_k], sv, sem_s)
        cp_d = pltpu.async_copy(adst_hbm.at[idx_d.at[k]], dv, sem_d)
        return cp_r, cp_s, cp_d

    def _drain_sc(b, k):
        id_k = idx_d.at[k]
        pltpu.make_async_copy(rows_o.at[b], num_sh.at[id_k],
                              sem_w[b]).wait()
        pltpu.make_async_copy(wv.at[b], den_sh.at[id_k],
                              sem_v[b]).wait()

    def _process(b, k, drain_k):
        id_k = idx_d.at[k]
        cp_r, cp_s, cp_d = _gather(k)
        cp_s.wait()
        cp_d.wait()
        wis = []
        for g in range(CH // 16):
            z = sv[pl.ds(g * 16, 16)] + dv[pl.ds(g * 16, 16)]
            wq = jnp.exp(jnp.maximum(z, 0.2 * z))
            wis.append(wq)
        if drain_k is not None:
            _drain_sc(b, drain_k)
        for g in range(CH // 16):
            wv[b, pl.ds(g * 16, 16)] = wis[g]
        cp_r.wait()
        for g in range(CH // 16):
            for i in range(16):
                wi = wis[g][i]
                r = g * 16 + i
                rr = rows.at[r]
                ro = rows_o.at[b].at[r]
                for j in range(HID // 16):
                    ro[pl.ds(j * 16, 16)] = rr[pl.ds(j * 16, 16)] * wi
        pltpu.async_copy(rows_o.at[b], num_sh.at[id_k], sem_w[b],
                         add=True)
        pltpu.async_copy(wv.at[b], den_sh.at[id_k], sem_v[b], add=True)

    def _pair(g, _):
        _process(0, 2 * g, 2 * g - 2)
        _process(1, 2 * g + 1, 2 * g - 1)
        return 0

    for h0, hn in ((0, H1), (H1, H2)):
        pltpu.sync_copy(src_hbm.at[w].at[pl.ds(h0, hn)],
                        idx_s.at[pl.ds(0, hn)])
        pltpu.sync_copy(dst_hbm.at[w].at[pl.ds(h0, hn)],
                        idx_d.at[pl.ds(0, hn)])
        _process(0, 0, None)
        _process(1, 1, None)
        lax.fori_loop(1, hn // 2, _pair, 0)
        if hn % 2:
            _process(0, hn - 1, hn - 3)
            _drain_sc(0, hn - 1)
            _drain_sc(1, hn - 2)
        else:
            _drain_sc(0, hn - 2)
            _drain_sc(1, hn - 1)
    plsc.subcore_barrier()

    woff = s * 624
    pltpu.sync_copy(num_sh.at[pl.ds(woff, 624)],
                    nump_hbm.at[c].at[pl.ds(woff, 624)])
    pltpu.sync_copy(den_sh.at[pl.ds(s * 640, 640)],
                    denp_hbm.at[c].at[pl.ds(s * 640, 640)])

    @pl.when(s == NS - 1)
    def _():
        pltpu.sync_copy(num_sh.at[pl.ds(NS * 624, N - NS * 624)],
                        nump_hbm.at[c].at[pl.ds(NS * 624, N - NS * 624)])


def _sc_edge(h, asrc, adst, src, dst):
    mesh = plsc.VectorSubcoreMesh(core_axis_name="c", subcore_axis_name="s",
                                  num_cores=NC, num_subcores=NS)
    f = pl.kernel(
        _sc_edge_body,
        out_type=[
            jax.ShapeDtypeStruct((NC, N, HID), jnp.float32),
            jax.ShapeDtypeStruct((NC, DEN_PAD), jnp.float32),
        ],
        mesh=mesh,
        scratch_types=[
            pltpu.VMEM_SHARED((N, HID), jnp.float32),
            pltpu.VMEM_SHARED((DEN_PAD,), jnp.float32),
            pltpu.VMEM((H1, CH), jnp.int32),
            pltpu.VMEM((H1, CH), jnp.int32),
            pltpu.VMEM((CH,), jnp.float32),
            pltpu.VMEM((CH,), jnp.float32),
            pltpu.VMEM((2, CH), jnp.float32),
            pltpu.VMEM((CH, HID), jnp.float32),
            pltpu.VMEM((2, CH, HID), jnp.float32),
            pltpu.VMEM((640,), jnp.float32),
        ] + [pltpu.SemaphoreType.DMA] * 7,
    )
    return f(h, asrc, adst, src, dst)


def kernel(x, edge_index, batch, W1, att_src1, att_dst1, b1,
           W2, att_src2, att_dst2, b2, fc1_W, fc1_b, fc2_W, fc2_b):
    src = edge_index[0].reshape(NW, NCHUNK, CH)
    dst = edge_index[1].reshape(NW, NCHUNK, CH)

    h1, num01, asrc1, adst1, den01 = _tc_head(
        x, W1, att_src1.reshape(HID, 1), att_dst1.reshape(HID, 1))
    numP1, denP1 = _sc_edge(h1, asrc1.reshape(N), adst1.reshape(N), src, dst)
    denP1 = denP1[:, :N].reshape(NC, N, 1)

    h2, num02, asrc2, adst2, den02 = _tc_mid(
        numP1, denP1, num01, den01, b1.reshape(1, HID),
        W2, att_src2.reshape(HID, 1), att_dst2.reshape(HID, 1))
    numP2, denP2 = _sc_edge(h2, asrc2.reshape(N), adst2.reshape(N), src, dst)
    denP2 = denP2[:, :N].reshape(NC, N, 1)

    return _tc_final(
        numP2, denP2, num02, den02, b2.reshape(1, HID),
        batch.reshape(N, 1), fc1_W, fc1_b.reshape(1, NHID),
        fc2_W, fc2_b.reshape(1, NCLS))
